# baseline (speedup 1.0000x reference)
import functools

import jax
import jax.numpy as jnp
from jax import lax
from jax.experimental import pallas as pl
from jax.experimental.pallas import tpu as pltpu
from jax.experimental.pallas import tpu_sc as plsc

D_MODEL = 1024
BATCH = 4
SEQ = 4096
LANES = 16

NC = 2
NS = 16
NW = NC * NS
S_PER_W = SEQ // NW
CH = 16
NCHUNK = S_PER_W // CH


def _add_pe(rows_ref, pe_ref):
    def row_body(r, carry):
        def col_body(qq, carry2):
            sl = pl.ds(qq * LANES, LANES)
            rows_ref[r, sl] = rows_ref[r, sl] + pe_ref[r, sl]
            return carry2
        return lax.fori_loop(0, D_MODEL // LANES, col_body, carry, unroll=8)
    lax.fori_loop(0, CH, row_body, 0)


def _body(x_hbm, pe_hbm, table_hbm, out_hbm,
          idx_all, r0, r1, r2, r3, pb0, pb1,
          sg0, sg1, sg2, sg3, so0, so1, so2, so3, sp0, sp1):
    rows = (r0, r1, r2, r3)
    pe_b = (pb0, pb1)
    sem_g = (sg0, sg1, sg2, sg3)
    sem_o = (so0, so1, so2, so3)
    sem_p = (sp0, sp1)

    wid = lax.axis_index("s") * NC + lax.axis_index("c")
    s_base = wid * S_PER_W

    for b in range(BATCH):
        pltpu.sync_copy(x_hbm.at[b, pl.ds(s_base, S_PER_W)], idx_all.at[b])

    def pe_start(j, g):
        return pltpu.async_copy(pe_hbm.at[pl.ds(s_base + j * CH, CH)],
                                pe_b[g], sem_p[g])

    def gather_start(b, j, p):
        return pltpu.async_copy(
            table_hbm.at[idx_all.at[b, pl.ds(j * CH, CH)]],
            rows[p], sem_g[p])

    def gather_wait(b, j, p):
        pltpu.make_async_copy(
            table_hbm.at[idx_all.at[b, pl.ds(j * CH, CH)]],
            rows[p], sem_g[p]).wait()

    def out_start(b, j, p):
        return pltpu.async_copy(
            rows[p], out_hbm.at[b, pl.ds(s_base + j * CH, CH)], sem_o[p])

    def out_wait(p):
        pltpu.make_async_copy(
            rows[p], out_hbm.at[0, pl.ds(s_base, CH)], sem_o[p]).wait()

    pe_start(0, 0)
    gather_start(0, 0, 0)
    gather_start(1, 0, 1)

    def pair(jj, carry):
        for g in range(2):
            j = 2 * jj + g
            pltpu.make_async_copy(pe_hbm.at[pl.ds(s_base + j * CH, CH)],
                                  pe_b[g], sem_p[g]).wait()
            if g == 0:
                pe_start(j + 1, 1)
            else:
                @pl.when(jj < NCHUNK // 2 - 1)
                def _():
                    pe_start(j + 1, 0)
            for b in range(BATCH):
                gather_wait(b, j, b)
                if b < 2:
                    if g == 0:
                        @pl.when(jj > 0)
                        def _():
                            out_wait(b + 2)
                    else:
                        out_wait(b + 2)
                    gather_start(b + 2, j, b + 2)
                else:
                    out_wait(b - 2)
                    if g == 0:
                        gather_start(b - 2, j + 1, b - 2)
                    else:
                        @pl.when(jj < NCHUNK // 2 - 1)
                        def _():
                            gather_start(b - 2, j + 1, b - 2)
                _add_pe(rows[b], pe_b[g])
                out_start(b, j, b)
        return carry

    lax.fori_loop(0, NCHUNK // 2, pair, 0)
    out_wait(2)
    out_wait(3)


@functools.partial(jax.jit, static_argnames=())
def kernel(x, table, pe):
    xi = x.astype(jnp.int32)
    pe2 = pe.reshape(pe.shape[1], pe.shape[2])
    mesh = plsc.VectorSubcoreMesh(core_axis_name="c", subcore_axis_name="s")
    run = functools.partial(
        pl.kernel,
        out_type=jax.ShapeDtypeStruct((BATCH, SEQ, D_MODEL), jnp.float32),
        mesh=mesh,
        scratch_types=[
            pltpu.VMEM((BATCH, S_PER_W), jnp.int32),
            pltpu.VMEM((CH, D_MODEL), jnp.float32),
            pltpu.VMEM((CH, D_MODEL), jnp.float32),
            pltpu.VMEM((CH, D_MODEL), jnp.float32),
            pltpu.VMEM((CH, D_MODEL), jnp.float32),
            pltpu.VMEM((CH, D_MODEL), jnp.float32),
            pltpu.VMEM((CH, D_MODEL), jnp.float32),
        ] + [pltpu.SemaphoreType.DMA] * 10,
    )(_body)
    return run(xi, pe2, table)


# speedup vs baseline: 1.7564x; 1.7564x over previous
import functools

import jax
import jax.numpy as jnp
from jax import lax
from jax.experimental import pallas as pl
from jax.experimental.pallas import tpu as pltpu
from jax.experimental.pallas import tpu_sc as plsc

D_MODEL = 1024
BATCH = 4
SEQ = 4096
LANES = 16

NC = 2
NS = 16
NW = NC * NS
S_PER_W = SEQ // NW
CH = 16
NCHUNK = S_PER_W // CH


def _add_pe(rows_ref, pe_ref):
    def row_body(r, carry):
        def col_body(qq, carry2):
            sl = pl.ds(qq * LANES, LANES)
            rows_ref[r, sl] = rows_ref[r, sl] + pe_ref[r, sl]
            return carry2
        return lax.fori_loop(0, D_MODEL // LANES, col_body, carry, unroll=2)
    lax.fori_loop(0, CH, row_body, 0)


def _body(x_hbm, pe_hbm, table_hbm, out_hbm,
          idx_all, r0, r1, r2, r3, pb0, pb1,
          sg0, sg1, sg2, sg3, so0, so1, so2, so3, sp0, sp1):
    rows = (r0, r1, r2, r3)
    pe_b = (pb0, pb1)
    sem_g = (sg0, sg1, sg2, sg3)
    sem_o = (so0, so1, so2, so3)
    sem_p = (sp0, sp1)

    wid = lax.axis_index("s") * NC + lax.axis_index("c")
    s_base = wid * S_PER_W

    for b in range(BATCH):
        pltpu.sync_copy(x_hbm.at[b, pl.ds(s_base, S_PER_W)], idx_all.at[b])

    def pe_start(j, g):
        return pltpu.async_copy(pe_hbm.at[pl.ds(s_base + j * CH, CH)],
                                pe_b[g], sem_p[g])

    def gather_start(b, j, p):
        return pltpu.async_copy(
            table_hbm.at[idx_all.at[b, pl.ds(j * CH, CH)]],
            rows[p], sem_g[p])

    def gather_wait(b, j, p):
        pltpu.make_async_copy(
            table_hbm.at[idx_all.at[b, pl.ds(j * CH, CH)]],
            rows[p], sem_g[p]).wait()

    def out_start(b, j, p):
        return pltpu.async_copy(
            rows[p], out_hbm.at[b, pl.ds(s_base + j * CH, CH)], sem_o[p])

    def out_wait(p):
        pltpu.make_async_copy(
            rows[p], out_hbm.at[0, pl.ds(s_base, CH)], sem_o[p]).wait()

    pe_start(0, 0)
    gather_start(0, 0, 0)
    gather_start(1, 0, 1)

    def pair(jj, carry):
        for g in range(2):
            j = 2 * jj + g
            pltpu.make_async_copy(pe_hbm.at[pl.ds(s_base + j * CH, CH)],
                                  pe_b[g], sem_p[g]).wait()
            if g == 0:
                pe_start(j + 1, 1)
            else:
                @pl.when(jj < NCHUNK // 2 - 1)
                def _():
                    pe_start(j + 1, 0)
            for b in range(BATCH):
                gather_wait(b, j, b)
                if b < 2:
                    if g == 0:
                        @pl.when(jj > 0)
                        def _():
                            out_wait(b + 2)
                    else:
                        out_wait(b + 2)
                    gather_start(b + 2, j, b + 2)
                else:
                    out_wait(b - 2)
                    if g == 0:
                        gather_start(b - 2, j + 1, b - 2)
                    else:
                        @pl.when(jj < NCHUNK // 2 - 1)
                        def _():
                            gather_start(b - 2, j + 1, b - 2)
                _add_pe(rows[b], pe_b[g])
                out_start(b, j, b)
        return carry

    lax.fori_loop(0, NCHUNK // 2, pair, 0)
    out_wait(2)
    out_wait(3)


@functools.partial(jax.jit, static_argnames=())
def kernel(x, table, pe):
    xi = x.astype(jnp.int32)
    pe2 = pe.reshape(pe.shape[1], pe.shape[2])
    mesh = plsc.VectorSubcoreMesh(core_axis_name="c", subcore_axis_name="s")
    run = functools.partial(
        pl.kernel,
        out_type=jax.ShapeDtypeStruct((BATCH, SEQ, D_MODEL), jnp.float32),
        mesh=mesh,
        scratch_types=[
            pltpu.VMEM((BATCH, S_PER_W), jnp.int32),
            pltpu.VMEM((CH, D_MODEL), jnp.float32),
            pltpu.VMEM((CH, D_MODEL), jnp.float32),
            pltpu.VMEM((CH, D_MODEL), jnp.float32),
            pltpu.VMEM((CH, D_MODEL), jnp.float32),
            pltpu.VMEM((CH, D_MODEL), jnp.float32),
            pltpu.VMEM((CH, D_MODEL), jnp.float32),
        ] + [pltpu.SemaphoreType.DMA] * 10,
    )(_body)
    return run(xi, pe2, table)


# speedup vs baseline: 2.3777x; 1.3537x over previous
import functools

import jax
import jax.numpy as jnp
from jax import lax
from jax.experimental import pallas as pl
from jax.experimental.pallas import tpu as pltpu
from jax.experimental.pallas import tpu_sc as plsc

D_MODEL = 1024
BATCH = 4
SEQ = 4096
LANES = 16

NC = 2
NS = 16
NW = NC * NS
S_PER_W = SEQ // NW
CH = 16
NCHUNK = S_PER_W // CH


def _add_pe(rows_ref, pe_ref):
    @plsc.parallel_loop(0, CH)
    def _(r):
        def col_body(qq, carry2):
            sl = pl.ds(qq * LANES, LANES)
            rows_ref[r, sl] = rows_ref[r, sl] + pe_ref[r, sl]
            return carry2
        lax.fori_loop(0, D_MODEL // LANES, col_body, 0, unroll=4)


def _body(x_hbm, pe_hbm, table_hbm, out_hbm,
          idx_all, r0, r1, r2, r3, pb0, pb1,
          sg0, sg1, sg2, sg3, so0, so1, so2, so3, sp0, sp1):
    rows = (r0, r1, r2, r3)
    pe_b = (pb0, pb1)
    sem_g = (sg0, sg1, sg2, sg3)
    sem_o = (so0, so1, so2, so3)
    sem_p = (sp0, sp1)

    wid = lax.axis_index("s") * NC + lax.axis_index("c")
    s_base = wid * S_PER_W

    for b in range(BATCH):
        pltpu.sync_copy(x_hbm.at[b, pl.ds(s_base, S_PER_W)], idx_all.at[b])

    def pe_start(j, g):
        return pltpu.async_copy(pe_hbm.at[pl.ds(s_base + j * CH, CH)],
                                pe_b[g], sem_p[g])

    def gather_start(b, j, p):
        return pltpu.async_copy(
            table_hbm.at[idx_all.at[b, pl.ds(j * CH, CH)]],
            rows[p], sem_g[p])

    def gather_wait(b, j, p):
        pltpu.make_async_copy(
            table_hbm.at[idx_all.at[b, pl.ds(j * CH, CH)]],
            rows[p], sem_g[p]).wait()

    def out_start(b, j, p):
        return pltpu.async_copy(
            rows[p], out_hbm.at[b, pl.ds(s_base + j * CH, CH)], sem_o[p])

    def out_wait(p):
        pltpu.make_async_copy(
            rows[p], out_hbm.at[0, pl.ds(s_base, CH)], sem_o[p]).wait()

    pe_start(0, 0)
    gather_start(0, 0, 0)
    gather_start(1, 0, 1)

    def pair(jj, carry):
        for g in range(2):
            j = 2 * jj + g
            pltpu.make_async_copy(pe_hbm.at[pl.ds(s_base + j * CH, CH)],
                                  pe_b[g], sem_p[g]).wait()
            if g == 0:
                pe_start(j + 1, 1)
            else:
                @pl.when(jj < NCHUNK // 2 - 1)
                def _():
                    pe_start(j + 1, 0)
            for b in range(BATCH):
                gather_wait(b, j, b)
                if b < 2:
                    if g == 0:
                        @pl.when(jj > 0)
                        def _():
                            out_wait(b + 2)
                    else:
                        out_wait(b + 2)
                    gather_start(b + 2, j, b + 2)
                else:
                    out_wait(b - 2)
                    if g == 0:
                        gather_start(b - 2, j + 1, b - 2)
                    else:
                        @pl.when(jj < NCHUNK // 2 - 1)
                        def _():
                            gather_start(b - 2, j + 1, b - 2)
                _add_pe(rows[b], pe_b[g])
                out_start(b, j, b)
        return carry

    lax.fori_loop(0, NCHUNK // 2, pair, 0)
    out_wait(2)
    out_wait(3)


@functools.partial(jax.jit, static_argnames=())
def kernel(x, table, pe):
    xi = x.astype(jnp.int32)
    pe2 = pe.reshape(pe.shape[1], pe.shape[2])
    mesh = plsc.VectorSubcoreMesh(core_axis_name="c", subcore_axis_name="s")
    run = functools.partial(
        pl.kernel,
        out_type=jax.ShapeDtypeStruct((BATCH, SEQ, D_MODEL), jnp.float32),
        mesh=mesh,
        scratch_types=[
            pltpu.VMEM((BATCH, S_PER_W), jnp.int32),
            pltpu.VMEM((CH, D_MODEL), jnp.float32),
            pltpu.VMEM((CH, D_MODEL), jnp.float32),
            pltpu.VMEM((CH, D_MODEL), jnp.float32),
            pltpu.VMEM((CH, D_MODEL), jnp.float32),
            pltpu.VMEM((CH, D_MODEL), jnp.float32),
            pltpu.VMEM((CH, D_MODEL), jnp.float32),
        ] + [pltpu.SemaphoreType.DMA] * 10,
    )(_body)
    return run(xi, pe2, table)


# speedup vs baseline: 2.3835x; 1.0025x over previous
import functools

import jax
import jax.numpy as jnp
from jax import lax
from jax.experimental import pallas as pl
from jax.experimental.pallas import tpu as pltpu
from jax.experimental.pallas import tpu_sc as plsc

D_MODEL = 1024
BATCH = 4
SEQ = 4096
LANES = 16

NC = 2
NS = 16
NW = NC * NS
S_PER_W = SEQ // NW
CH = 16
NCHUNK = S_PER_W // CH


def _add_pe(rows_ref, pe_ref):
    def row_body(r, carry):
        def col_body(qq, carry2):
            sl = pl.ds(qq * LANES, LANES)
            rows_ref[r, sl] = rows_ref[r, sl] + pe_ref[r, sl]
            return carry2
        return lax.fori_loop(0, D_MODEL // LANES, col_body, carry, unroll=4)
    lax.fori_loop(0, CH, row_body, 0)


def _body(x_hbm, pe_hbm, table_hbm, out_hbm,
          idx_all, r0, r1, r2, r3, pb0, pb1,
          sg0, sg1, sg2, sg3, so0, so1, so2, so3, sp0, sp1):
    rows = (r0, r1, r2, r3)
    pe_b = (pb0, pb1)
    sem_g = (sg0, sg1, sg2, sg3)
    sem_o = (so0, so1, so2, so3)
    sem_p = (sp0, sp1)

    wid = lax.axis_index("s") * NC + lax.axis_index("c")
    s_base = wid * S_PER_W

    for b in range(BATCH):
        pltpu.sync_copy(x_hbm.at[b, pl.ds(s_base, S_PER_W)], idx_all.at[b])

    def pe_start(j, g):
        return pltpu.async_copy(pe_hbm.at[pl.ds(s_base + j * CH, CH)],
                                pe_b[g], sem_p[g])

    def gather_start(b, j, p):
        return pltpu.async_copy(
            table_hbm.at[idx_all.at[b, pl.ds(j * CH, CH)]],
            rows[p], sem_g[p])

    def gather_wait(b, j, p):
        pltpu.make_async_copy(
            table_hbm.at[idx_all.at[b, pl.ds(j * CH, CH)]],
            rows[p], sem_g[p]).wait()

    def out_start(b, j, p):
        return pltpu.async_copy(
            rows[p], out_hbm.at[b, pl.ds(s_base + j * CH, CH)], sem_o[p])

    def out_wait(p):
        pltpu.make_async_copy(
            rows[p], out_hbm.at[0, pl.ds(s_base, CH)], sem_o[p]).wait()

    pe_start(0, 0)
    gather_start(0, 0, 0)
    gather_start(1, 0, 1)

    def pair(jj, carry):
        for g in range(2):
            j = 2 * jj + g
            pltpu.make_async_copy(pe_hbm.at[pl.ds(s_base + j * CH, CH)],
                                  pe_b[g], sem_p[g]).wait()
            if g == 0:
                pe_start(j + 1, 1)
            else:
                @pl.when(jj < NCHUNK // 2 - 1)
                def _():
                    pe_start(j + 1, 0)
            for b in range(BATCH):
                gather_wait(b, j, b)
                if b < 2:
                    if g == 0:
                        @pl.when(jj > 0)
                        def _():
                            out_wait(b + 2)
                    else:
                        out_wait(b + 2)
                    gather_start(b + 2, j, b + 2)
                else:
                    out_wait(b - 2)
                    if g == 0:
                        gather_start(b - 2, j + 1, b - 2)
                    else:
                        @pl.when(jj < NCHUNK // 2 - 1)
                        def _():
                            gather_start(b - 2, j + 1, b - 2)
                _add_pe(rows[b], pe_b[g])
                out_start(b, j, b)
        return carry

    lax.fori_loop(0, NCHUNK // 2, pair, 0)
    out_wait(2)
    out_wait(3)


@functools.partial(jax.jit, static_argnames=())
def kernel(x, table, pe):
    xi = x.astype(jnp.int32)
    pe2 = pe.reshape(pe.shape[1], pe.shape[2])
    mesh = plsc.VectorSubcoreMesh(core_axis_name="c", subcore_axis_name="s")
    run = functools.partial(
        pl.kernel,
        out_type=jax.ShapeDtypeStruct((BATCH, SEQ, D_MODEL), jnp.float32),
        mesh=mesh,
        scratch_types=[
            pltpu.VMEM((BATCH, S_PER_W), jnp.int32),
            pltpu.VMEM((CH, D_MODEL), jnp.float32),
            pltpu.VMEM((CH, D_MODEL), jnp.float32),
            pltpu.VMEM((CH, D_MODEL), jnp.float32),
            pltpu.VMEM((CH, D_MODEL), jnp.float32),
            pltpu.VMEM((CH, D_MODEL), jnp.float32),
            pltpu.VMEM((CH, D_MODEL), jnp.float32),
        ] + [pltpu.SemaphoreType.DMA] * 10,
    )(_body)
    return run(xi, pe2, table)
